# Initial kernel scaffold; baseline (speedup 1.0000x reference)
import jax
import jax.numpy as jnp
from jax.experimental import pallas as pl


def kernel(x):
    raise NotImplementedError("write your pallas kernel here")


# speedup vs baseline: 576.3678x; 576.3678x over previous
import functools

import jax
import jax.numpy as jnp
from jax import lax
from jax.experimental import pallas as pl
from jax.experimental.pallas import tpu as pltpu
from jax.experimental.pallas import tpu_sc as plsc

ROWS, COLS = 8192, 2048
LANES = 16
NUM_CORES = 2
NUM_SUBCORES = 16
NW = NUM_CORES * NUM_SUBCORES
ROWS_PER_W = ROWS // NW
CHUNK_ROWS = 16
CHUNK_ELEMS = CHUNK_ROWS * COLS
STEPS = ROWS_PER_W // CHUNK_ROWS
SLICES_PER_ROW = COLS // LANES


def _csr_check_body(x_hbm, out_hbm, buf, mism_v):
    c = lax.axis_index("c")
    s = lax.axis_index("s")
    wid = s * NUM_CORES + c
    base = wid * (ROWS_PER_W * COLS)

    lane = lax.iota(jnp.int32, LANES)
    zi = jnp.zeros((LANES,), jnp.int32)
    zf = jnp.zeros((LANES,), jnp.float32)

    def step_fn(step, mism):
        pltpu.sync_copy(x_hbm.at[pl.ds(base + step * CHUNK_ELEMS, CHUNK_ELEMS)], buf)

        def row_fn(r, mism):
            row_global = wid * ROWS_PER_W + step * CHUNK_ROWS + r
            rowbase = row_global * COLS
            row_off = r * COLS

            def slice_fn(j, carry):
                cnt_i, cnt_f, col_i, col_o, val_a, val_b = carry
                xs = buf[pl.ds(row_off + j * LANES, LANES)]
                m = xs > 0.0
                col = lane + j * LANES
                colo = lax.bitwise_and(col + rowbase, COLS - 1)
                cnt_i = cnt_i + jnp.where(m, 1, 0)
                cnt_f = cnt_f + jnp.where(m, 1.0, 0.0)
                col_i = col_i + jnp.where(m, col, 0)
                col_o = col_o + jnp.where(m, colo, 0)
                val_a = val_a + jnp.maximum(xs, 0.0)
                val_b = val_b + jnp.where(m, xs, 0.0)
                return (cnt_i, cnt_f, col_i, col_o, val_a, val_b)

            cnt_i, cnt_f, col_i, col_o, val_a, val_b = lax.fori_loop(
                0, SLICES_PER_ROW, slice_fn, (zi, zf, zi, zi, zf, zf))
            bad = ((cnt_i != cnt_f.astype(jnp.int32))
                   | (col_i != col_o)
                   | (val_a != val_b))
            return mism + jnp.where(bad, 1, 0)

        return lax.fori_loop(0, CHUNK_ROWS, row_fn, mism)

    mism = lax.fori_loop(0, STEPS, step_fn, zi)
    mism_v[...] = mism
    pltpu.sync_copy(mism_v, out_hbm.at[wid])


@functools.partial(jax.jit, static_argnames=())
def _csr_check(xf):
    mesh = plsc.VectorSubcoreMesh(core_axis_name="c", subcore_axis_name="s")
    run = functools.partial(
        pl.kernel,
        out_type=jax.ShapeDtypeStruct((NW, LANES), jnp.int32),
        mesh=mesh,
        scratch_types=[
            pltpu.VMEM((CHUNK_ELEMS,), jnp.float32),
            pltpu.VMEM((LANES,), jnp.int32),
        ],
    )(_csr_check_body)
    return run(xf)


def kernel(x):
    mism = _csr_check(x.reshape(-1))
    return jnp.sum(mism) == 0


# speedup vs baseline: 858.2305x; 1.4890x over previous
import functools

import jax
import jax.numpy as jnp
from jax import lax
from jax.experimental import pallas as pl
from jax.experimental.pallas import tpu as pltpu
from jax.experimental.pallas import tpu_sc as plsc

ROWS, COLS = 8192, 2048
LANES = 16
NUM_CORES = 2
NUM_SUBCORES = 16
NW = NUM_CORES * NUM_SUBCORES
ROWS_PER_W = ROWS // NW
CHUNK_ROWS = 16
STEPS = ROWS_PER_W // CHUNK_ROWS
SLICES_PER_ROW = COLS // LANES


def _csr_check_body(x_hbm, out_hbm, buf0, buf1, mism_v, sem0, sem1):
    c = lax.axis_index("c")
    s = lax.axis_index("s")
    wid = s * NUM_CORES + c
    row0 = wid * ROWS_PER_W

    lane = lax.iota(jnp.int32, LANES)
    zi = jnp.zeros((LANES,), jnp.int32)
    zf = jnp.zeros((LANES,), jnp.float32)

    def copy_in(step, buf, sem):
        return pltpu.make_async_copy(
            x_hbm.at[pl.ds(row0 + step * CHUNK_ROWS, CHUNK_ROWS)], buf, sem)

    def do_chunk(step, buf, mism):
        def row_fn(r, mism):
            rowbase = (row0 + step * CHUNK_ROWS + r) * COLS

            def slice_fn(j, carry):
                cnt_a, cnt_b, col_d, val_a, val_b, col = carry
                xs = buf[r, pl.ds(j * LANES, LANES)]
                m = xs > 0.0
                colo = lax.bitwise_and(col + rowbase, COLS - 1)
                cnt_a = cnt_a + jnp.where(m, 1, 0)
                cnt_b = cnt_b + jnp.where(m, 1.0, 0.0)
                col_d = col_d + jnp.where(m, col - colo, 0)
                val_a = val_a + jnp.maximum(xs, 0.0)
                val_b = val_b + jnp.where(m, xs, 0.0)
                return (cnt_a, cnt_b, col_d, val_a, val_b, col + LANES)

            cnt_a, cnt_b, col_d, val_a, val_b, _ = plsc.parallel_loop(
                0, SLICES_PER_ROW, unroll=8,
                carry=(zi, zf, zi, zf, zf, lane))(slice_fn)
            bad = ((cnt_a != cnt_b.astype(jnp.int32))
                   | (col_d != 0)
                   | (val_a != val_b))
            return mism + jnp.where(bad, 1, 0)

        return lax.fori_loop(0, CHUNK_ROWS, row_fn, mism)

    copy_in(0, buf0, sem0).start()
    copy_in(1, buf1, sem1).start()

    def step_fn(p, mism):
        copy_in(2 * p, buf0, sem0).wait()
        mism = do_chunk(2 * p, buf0, mism)

        @pl.when(p < STEPS // 2 - 1)
        def _():
            copy_in(2 * p + 2, buf0, sem0).start()

        copy_in(2 * p + 1, buf1, sem1).wait()
        mism = do_chunk(2 * p + 1, buf1, mism)

        @pl.when(p < STEPS // 2 - 1)
        def _():
            copy_in(2 * p + 3, buf1, sem1).start()

        return mism

    mism = lax.fori_loop(0, STEPS // 2, step_fn, zi)
    mism_v[...] = mism
    pltpu.sync_copy(mism_v, out_hbm.at[wid])


@jax.jit
def _csr_check(x):
    mesh = plsc.VectorSubcoreMesh(core_axis_name="c", subcore_axis_name="s")
    run = functools.partial(
        pl.kernel,
        out_type=jax.ShapeDtypeStruct((NW, LANES), jnp.int32),
        mesh=mesh,
        scratch_types=[
            pltpu.VMEM((CHUNK_ROWS, COLS), jnp.float32),
            pltpu.VMEM((CHUNK_ROWS, COLS), jnp.float32),
            pltpu.VMEM((LANES,), jnp.int32),
            pltpu.SemaphoreType.DMA,
            pltpu.SemaphoreType.DMA,
        ],
    )(_csr_check_body)
    return run(x)


def kernel(x):
    mism = _csr_check(x)
    return jnp.sum(mism) == 0


# speedup vs baseline: 1237.3922x; 1.4418x over previous
import functools

import jax
import jax.numpy as jnp
from jax import lax
from jax.experimental import pallas as pl
from jax.experimental.pallas import tpu as pltpu
from jax.experimental.pallas import tpu_sc as plsc

ROWS, COLS = 8192, 2048
LANES = 16
NUM_CORES = 2
NUM_SUBCORES = 16
NW = NUM_CORES * NUM_SUBCORES
ROWS_PER_W = ROWS // NW
CHUNK_ROWS = 16
STEPS = ROWS_PER_W // CHUNK_ROWS
UNROLL = 8
GROUPS_PER_ROW = COLS // (LANES * UNROLL)


def _csr_check_body(x_hbm, out_hbm, buf0, buf1, mism_v, sem0, sem1):
    c = lax.axis_index("c")
    s = lax.axis_index("s")
    wid = s * NUM_CORES + c
    row0 = wid * ROWS_PER_W

    lane = lax.iota(jnp.int32, LANES)
    zi = jnp.zeros((LANES,), jnp.int32)
    zf = jnp.zeros((LANES,), jnp.float32)

    def copy_in(step, buf, sem):
        return pltpu.make_async_copy(
            x_hbm.at[pl.ds(row0 + step * CHUNK_ROWS, CHUNK_ROWS)], buf, sem)

    def do_chunk(step, buf, mism):
        def row_fn(r, mism):
            rowbase = (row0 + step * CHUNK_ROWS + r) * COLS

            def group_fn(g, carry):
                cnt_a, cnt_b, col_d, val_a, val_b = carry
                gbase = g * (LANES * UNROLL)
                for k in range(UNROLL):
                    xs = buf[r, pl.ds(gbase + k * LANES, LANES)]
                    m = xs > 0.0
                    col = lane + (gbase + k * LANES)
                    colo = lax.bitwise_and(col + rowbase, COLS - 1)
                    cnt_a = cnt_a + jnp.where(m, 1, 0)
                    cnt_b = cnt_b + jnp.where(m, 1.0, 0.0)
                    col_d = col_d + jnp.where(m, col - colo, 0)
                    val_a = val_a + jnp.maximum(xs, 0.0)
                    val_b = val_b + jnp.where(m, xs, 0.0)
                return (cnt_a, cnt_b, col_d, val_a, val_b)

            cnt_a, cnt_b, col_d, val_a, val_b = plsc.parallel_loop(
                0, GROUPS_PER_ROW, carry=(zi, zf, zi, zf, zf))(group_fn)
            bad = ((cnt_a != cnt_b.astype(jnp.int32))
                   | (col_d != 0)
                   | (val_a != val_b))
            return mism + jnp.where(bad, 1, 0)

        return lax.fori_loop(0, CHUNK_ROWS, row_fn, mism)

    copy_in(0, buf0, sem0).start()
    copy_in(1, buf1, sem1).start()

    def step_fn(p, mism):
        copy_in(2 * p, buf0, sem0).wait()
        mism = do_chunk(2 * p, buf0, mism)

        @pl.when(p < STEPS // 2 - 1)
        def _():
            copy_in(2 * p + 2, buf0, sem0).start()

        copy_in(2 * p + 1, buf1, sem1).wait()
        mism = do_chunk(2 * p + 1, buf1, mism)

        @pl.when(p < STEPS // 2 - 1)
        def _():
            copy_in(2 * p + 3, buf1, sem1).start()

        return mism

    mism = lax.fori_loop(0, STEPS // 2, step_fn, zi)
    mism_v[...] = mism
    pltpu.sync_copy(mism_v, out_hbm.at[wid])


@jax.jit
def _csr_check(x):
    mesh = plsc.VectorSubcoreMesh(core_axis_name="c", subcore_axis_name="s")
    run = functools.partial(
        pl.kernel,
        out_type=jax.ShapeDtypeStruct((NW, LANES), jnp.int32),
        mesh=mesh,
        scratch_types=[
            pltpu.VMEM((CHUNK_ROWS, COLS), jnp.float32),
            pltpu.VMEM((CHUNK_ROWS, COLS), jnp.float32),
            pltpu.VMEM((LANES,), jnp.int32),
            pltpu.SemaphoreType.DMA,
            pltpu.SemaphoreType.DMA,
        ],
    )(_csr_check_body)
    return run(x)


def kernel(x):
    mism = _csr_check(x)
    return jnp.sum(mism) == 0


# speedup vs baseline: 2251.2192x; 1.8193x over previous
import functools

import jax
import jax.numpy as jnp
from jax import lax
from jax.experimental import pallas as pl
from jax.experimental.pallas import tpu as pltpu
from jax.experimental.pallas import tpu_sc as plsc

ROWS, COLS = 8192, 2048
LANES = 16

NUM_CORES = 2
NUM_SUBCORES = 16
NW = NUM_CORES * NUM_SUBCORES
SC_ROWS = 2048
ROWS_PER_W = SC_ROWS // NW
CHUNK_ROWS = 16
STEPS = ROWS_PER_W // CHUNK_ROWS
UNROLL = 8
GROUPS_PER_ROW = COLS // (LANES * UNROLL)

TC_BR = 256
TC_BLOCKS = (ROWS - SC_ROWS) // TC_BR


def _csr_check_sc_body(x_hbm, out_hbm, buf0, buf1, mism_v, sem0, sem1):
    c = lax.axis_index("c")
    s = lax.axis_index("s")
    wid = s * NUM_CORES + c
    row0 = wid * ROWS_PER_W

    lane = lax.iota(jnp.int32, LANES)
    zi = jnp.zeros((LANES,), jnp.int32)
    zf = jnp.zeros((LANES,), jnp.float32)

    def copy_in(step, buf, sem):
        return pltpu.make_async_copy(
            x_hbm.at[pl.ds(row0 + step * CHUNK_ROWS, CHUNK_ROWS)], buf, sem)

    def do_chunk(step, buf, mism):
        def row_fn(r, mism):
            rowbase = (row0 + step * CHUNK_ROWS + r) * COLS

            def group_fn(g, carry):
                cnt_a, cnt_b, col_d, val_a, val_b = carry
                gbase = g * (LANES * UNROLL)
                for k in range(UNROLL):
                    xs = buf[r, pl.ds(gbase + k * LANES, LANES)]
                    m = xs > 0.0
                    col = lane + (gbase + k * LANES)
                    colo = lax.bitwise_and(col + rowbase, COLS - 1)
                    cnt_a = cnt_a + jnp.where(m, 1, 0)
                    cnt_b = cnt_b + jnp.where(m, 1.0, 0.0)
                    col_d = col_d + jnp.where(m, col - colo, 0)
                    val_a = val_a + jnp.maximum(xs, 0.0)
                    val_b = val_b + jnp.where(m, xs, 0.0)
                return (cnt_a, cnt_b, col_d, val_a, val_b)

            cnt_a, cnt_b, col_d, val_a, val_b = plsc.parallel_loop(
                0, GROUPS_PER_ROW, carry=(zi, zf, zi, zf, zf))(group_fn)
            bad = ((cnt_a != cnt_b.astype(jnp.int32))
                   | (col_d != 0)
                   | (val_a != val_b))
            return mism + jnp.where(bad, 1, 0)

        return lax.fori_loop(0, CHUNK_ROWS, row_fn, mism)

    copy_in(0, buf0, sem0).start()
    copy_in(1, buf1, sem1).start()

    def step_fn(p, mism):
        copy_in(2 * p, buf0, sem0).wait()
        mism = do_chunk(2 * p, buf0, mism)

        @pl.when(p < STEPS // 2 - 1)
        def _():
            copy_in(2 * p + 2, buf0, sem0).start()

        copy_in(2 * p + 1, buf1, sem1).wait()
        mism = do_chunk(2 * p + 1, buf1, mism)

        @pl.when(p < STEPS // 2 - 1)
        def _():
            copy_in(2 * p + 3, buf1, sem1).start()

        return mism

    mism = lax.fori_loop(0, STEPS // 2, step_fn, zi)
    mism_v[...] = mism
    pltpu.sync_copy(mism_v, out_hbm.at[wid])


def _csr_check_sc(x):
    mesh = plsc.VectorSubcoreMesh(core_axis_name="c", subcore_axis_name="s")
    run = functools.partial(
        pl.kernel,
        out_type=jax.ShapeDtypeStruct((NW, LANES), jnp.int32),
        mesh=mesh,
        scratch_types=[
            pltpu.VMEM((CHUNK_ROWS, COLS), jnp.float32),
            pltpu.VMEM((CHUNK_ROWS, COLS), jnp.float32),
            pltpu.VMEM((LANES,), jnp.int32),
            pltpu.SemaphoreType.DMA,
            pltpu.SemaphoreType.DMA,
        ],
    )(_csr_check_sc_body)
    return run(x)


def _csr_check_tc_body(x_ref, out_ref):
    i = pl.program_id(0)
    x = x_ref[...]
    m = x > 0.0
    col = lax.broadcasted_iota(jnp.int32, (TC_BR, COLS), 1)
    rowg = (lax.broadcasted_iota(jnp.int32, (TC_BR, COLS), 0)
            + (SC_ROWS + i * TC_BR))
    colo = lax.bitwise_and(col + rowg * COLS, COLS - 1)
    cnt_a = jnp.sum(jnp.where(m, 1, 0), axis=1)
    cnt_b = jnp.sum(jnp.where(m, 1.0, 0.0), axis=1)
    col_d = jnp.sum(jnp.where(m, col - colo, 0), axis=1)
    val_a = jnp.sum(jnp.maximum(x, 0.0), axis=1)
    val_b = jnp.sum(jnp.where(m, x, 0.0), axis=1)
    bad = ((cnt_a != cnt_b.astype(jnp.int32))
           | (col_d != 0)
           | (val_a != val_b))
    out_ref[...] = jnp.where(bad, 1, 0).reshape(1, 1, TC_BR)


def _csr_check_tc(x):
    return pl.pallas_call(
        _csr_check_tc_body,
        grid=(TC_BLOCKS,),
        in_specs=[pl.BlockSpec((TC_BR, COLS),
                               lambda i: (i + SC_ROWS // TC_BR, 0))],
        out_specs=pl.BlockSpec((1, 1, TC_BR), lambda i: (i, 0, 0)),
        out_shape=jax.ShapeDtypeStruct((TC_BLOCKS, 1, TC_BR), jnp.int32),
    )(x)


@jax.jit
def _csr_check(x):
    sc_mism = _csr_check_sc(x)
    tc_bad = _csr_check_tc(x)
    return jnp.sum(sc_mism) + jnp.sum(tc_bad)


def kernel(x):
    return _csr_check(x) == 0


# speedup vs baseline: 2514.0573x; 1.1168x over previous
import functools

import jax
import jax.numpy as jnp
from jax import lax
from jax.experimental import pallas as pl
from jax.experimental.pallas import tpu as pltpu
from jax.experimental.pallas import tpu_sc as plsc

ROWS, COLS = 8192, 2048
LANES = 16

NUM_CORES = 2
NUM_SUBCORES = 16
NW = NUM_CORES * NUM_SUBCORES
SC_ROWS = 2048
ROWS_PER_W = SC_ROWS // NW
CHUNK_ROWS = 16
STEPS = ROWS_PER_W // CHUNK_ROWS
UNROLL = 8
GROUPS_PER_ROW = COLS // (LANES * UNROLL)

TC_BR = 512
TC_BLOCKS = (ROWS - SC_ROWS) // TC_BR


def _csr_check_sc_body(x_hbm, out_hbm, buf0, buf1, mism_v, sem0, sem1):
    c = lax.axis_index("c")
    s = lax.axis_index("s")
    wid = s * NUM_CORES + c
    row0 = wid * ROWS_PER_W

    lane = lax.iota(jnp.int32, LANES)
    zi = jnp.zeros((LANES,), jnp.int32)
    zf = jnp.zeros((LANES,), jnp.float32)

    def copy_in(step, buf, sem):
        return pltpu.make_async_copy(
            x_hbm.at[pl.ds(row0 + step * CHUNK_ROWS, CHUNK_ROWS)], buf, sem)

    def do_chunk(step, buf, mism):
        def row_fn(r, mism):
            rowbase = (row0 + step * CHUNK_ROWS + r) * COLS

            def group_fn(g, carry):
                cnt_a, cnt_b, col_d, val_a, val_b = carry
                gbase = g * (LANES * UNROLL)
                for k in range(UNROLL):
                    xs = buf[r, pl.ds(gbase + k * LANES, LANES)]
                    m = xs > 0.0
                    col = lane + (gbase + k * LANES)
                    colo = lax.bitwise_and(col + rowbase, COLS - 1)
                    cnt_a = cnt_a + jnp.where(m, 1, 0)
                    cnt_b = cnt_b + jnp.where(m, 1.0, 0.0)
                    col_d = col_d + jnp.where(m, col - colo, 0)
                    val_a = val_a + jnp.maximum(xs, 0.0)
                    val_b = val_b + jnp.where(m, xs, 0.0)
                return (cnt_a, cnt_b, col_d, val_a, val_b)

            cnt_a, cnt_b, col_d, val_a, val_b = plsc.parallel_loop(
                0, GROUPS_PER_ROW, carry=(zi, zf, zi, zf, zf))(group_fn)
            bad = ((cnt_a != cnt_b.astype(jnp.int32))
                   | (col_d != 0)
                   | (val_a != val_b))
            return mism + jnp.where(bad, 1, 0)

        return lax.fori_loop(0, CHUNK_ROWS, row_fn, mism)

    copy_in(0, buf0, sem0).start()
    copy_in(1, buf1, sem1).start()

    def step_fn(p, mism):
        copy_in(2 * p, buf0, sem0).wait()
        mism = do_chunk(2 * p, buf0, mism)

        @pl.when(p < STEPS // 2 - 1)
        def _():
            copy_in(2 * p + 2, buf0, sem0).start()

        copy_in(2 * p + 1, buf1, sem1).wait()
        mism = do_chunk(2 * p + 1, buf1, mism)

        @pl.when(p < STEPS // 2 - 1)
        def _():
            copy_in(2 * p + 3, buf1, sem1).start()

        return mism

    mism = lax.fori_loop(0, STEPS // 2, step_fn, zi)
    mism_v[...] = mism
    pltpu.sync_copy(mism_v, out_hbm.at[wid])


def _csr_check_sc(x):
    mesh = plsc.VectorSubcoreMesh(core_axis_name="c", subcore_axis_name="s")
    run = functools.partial(
        pl.kernel,
        out_type=jax.ShapeDtypeStruct((NW, LANES), jnp.int32),
        mesh=mesh,
        scratch_types=[
            pltpu.VMEM((CHUNK_ROWS, COLS), jnp.float32),
            pltpu.VMEM((CHUNK_ROWS, COLS), jnp.float32),
            pltpu.VMEM((LANES,), jnp.int32),
            pltpu.SemaphoreType.DMA,
            pltpu.SemaphoreType.DMA,
        ],
    )(_csr_check_sc_body)
    return run(x)


def _csr_check_tc_body(x_ref, out_ref):
    i = pl.program_id(0)
    x = x_ref[...]
    m = x > 0.0
    col2 = lax.broadcasted_iota(jnp.int32, (TC_BR, COLS), 1) + COLS
    rowg = (lax.broadcasted_iota(jnp.int32, (TC_BR, COLS), 0)
            + (SC_ROWS + i * TC_BR))
    flat = (col2 - COLS) + lax.shift_left(rowg, 11)
    colo2 = lax.bitwise_and(flat, COLS - 1) + COLS
    acc_i = jnp.sum(jnp.where(m, col2, 0), axis=1)
    acc_o = jnp.sum(jnp.where(m, colo2, 0), axis=1)
    val_a = jnp.sum(jnp.maximum(x, 0.0), axis=1)
    val_b = jnp.sum(jnp.where(m, x, 0.0), axis=1)
    bad = (acc_i != acc_o) | (val_a != val_b)
    out_ref[...] = jnp.where(bad, 1, 0).reshape(1, 1, TC_BR)


def _csr_check_tc(x):
    return pl.pallas_call(
        _csr_check_tc_body,
        grid=(TC_BLOCKS,),
        in_specs=[pl.BlockSpec((TC_BR, COLS),
                               lambda i: (i + SC_ROWS // TC_BR, 0))],
        out_specs=pl.BlockSpec((1, 1, TC_BR), lambda i: (i, 0, 0)),
        out_shape=jax.ShapeDtypeStruct((TC_BLOCKS, 1, TC_BR), jnp.int32),
    )(x)


@jax.jit
def _csr_check(x):
    sc_mism = _csr_check_sc(x)
    tc_bad = _csr_check_tc(x)
    return jnp.sum(sc_mism) + jnp.sum(tc_bad)


def kernel(x):
    return _csr_check(x) == 0
